# Initial kernel scaffold; baseline (speedup 1.0000x reference)
import jax
import jax.numpy as jnp
from jax.experimental import pallas as pl


def kernel(token_ids, embeddings):
    raise NotImplementedError("write your pallas kernel here")


# speedup vs baseline: 1.4768x; 1.4768x over previous
import functools

import jax
import jax.numpy as jnp
from jax import lax
from jax.experimental import pallas as pl
from jax.experimental.pallas import tpu as pltpu
from jax.experimental.pallas import tpu_sc as plsc

D = 32
NC, NS = 2, 16
NW = NC * NS
BATCH = 128
GRP = 8
STEPS = 200

_mesh = plsc.VectorSubcoreMesh(core_axis_name="c", subcore_axis_name="s")


@functools.partial(
    pl.kernel,
    out_type=jax.ShapeDtypeStruct((NW, STEPS, BATCH, D), jnp.float32),
    mesh=_mesh,
    compiler_params=pltpu.CompilerParams(use_tc_tiling_on_sc=False),
    scratch_types=[
        pltpu.VMEM((STEPS, BATCH), jnp.int32),
        pltpu.VMEM((GRP, BATCH, D), jnp.float32),
        pltpu.SemaphoreType.DMA,
    ],
)
def _emb_gather(idx_hbm, table_hbm, out_hbm, idx_v, rows_v, gsem):
    wid = lax.axis_index("s") * NC + lax.axis_index("c")
    pltpu.sync_copy(idx_hbm.at[wid], idx_v)

    def group(g, carry):
        waits = []
        for b in range(GRP):
            waits.append(
                pltpu.async_copy(
                    table_hbm.at[idx_v.at[g * GRP + b]], rows_v.at[b], gsem
                )
            )
        for w in waits:
            w.wait()
        pltpu.sync_copy(rows_v, out_hbm.at[wid, pl.ds(g * GRP, GRP)])
        return carry

    lax.fori_loop(0, STEPS // GRP, group, 0)


def kernel(token_ids, embeddings):
    b, s = token_ids.shape
    idx = token_ids.reshape(NW, STEPS, BATCH).astype(jnp.int32)
    out = _emb_gather(idx, embeddings)
    return out.reshape(b, s, D)


# speedup vs baseline: 1.4987x; 1.0148x over previous
import functools

import jax
import jax.numpy as jnp
from jax import lax
from jax.experimental import pallas as pl
from jax.experimental.pallas import tpu as pltpu
from jax.experimental.pallas import tpu_sc as plsc

D = 32
NC, NS = 2, 16
NW = NC * NS
BATCH = 128
GRP = 10
STEPS = 200
NGRP = STEPS // GRP
GB = GRP * BATCH

_mesh = plsc.VectorSubcoreMesh(core_axis_name="c", subcore_axis_name="s")


@functools.partial(
    pl.kernel,
    out_type=jax.ShapeDtypeStruct((NW, STEPS * BATCH, D), jnp.float32),
    mesh=_mesh,
    compiler_params=pltpu.CompilerParams(use_tc_tiling_on_sc=False),
    scratch_types=[
        pltpu.VMEM((STEPS, BATCH), jnp.int32),
        pltpu.VMEM((2, GB, D), jnp.float32),
        pltpu.SemaphoreType.DMA,
        pltpu.SemaphoreType.DMA,
        pltpu.SemaphoreType.DMA,
    ],
)
def _emb_gather(idx_hbm, table_hbm, out_hbm, idx_v, rows_v, gsem0, gsem1, osem):
    wid = lax.axis_index("s") * NC + lax.axis_index("c")
    pltpu.sync_copy(idx_hbm.at[wid], idx_v)
    gsems = (gsem0, gsem1)

    def fire(g, b):
        for k in range(GRP):
            pltpu.async_copy(
                table_hbm.at[idx_v.at[g * GRP + k]],
                rows_v.at[b, pl.ds(k * BATCH, BATCH)],
                gsems[b],
            )

    def drain_gathers(b):
        pltpu.make_async_copy(
            table_hbm.at[pl.ds(0, GB)], rows_v.at[b], gsems[b]
        ).wait()

    def start_out(g, b):
        pltpu.async_copy(
            rows_v.at[b], out_hbm.at[wid, pl.ds(g * GB, GB)], osem
        )

    def drain_out():
        pltpu.make_async_copy(
            rows_v.at[0], out_hbm.at[wid, pl.ds(0, GB)], osem
        ).wait()

    fire(0, 0)
    fire(1, 1)
    drain_gathers(0)
    start_out(0, 0)

    def pair(t, carry):
        g1 = 2 * t + 1
        drain_out()
        fire(g1 + 1, 0)
        drain_gathers(1)
        start_out(g1, 1)
        g2 = 2 * t + 2
        drain_out()
        fire(g2 + 1, 1)
        drain_gathers(0)
        start_out(g2, 0)
        return carry

    lax.fori_loop(0, (NGRP - 2) // 2, pair, 0)

    drain_out()
    drain_gathers(1)
    start_out(NGRP - 1, 1)
    drain_out()


def kernel(token_ids, embeddings):
    b, s = token_ids.shape
    idx = token_ids.reshape(NW, STEPS, BATCH).astype(jnp.int32)
    out = _emb_gather(idx, embeddings)
    return out.reshape(b, s, D)
